# baseline (speedup 1.0000x reference)
import functools

import jax
import jax.numpy as jnp
from jax import lax
from jax.experimental import pallas as pl
from jax.experimental.pallas import tpu as pltpu
from jax.experimental.pallas import tpu_sc as plsc

_S = 2048
_D = 2048
_TN = 256
_TMS = 128
_TMAX = 21
_PAD = _TMAX * _TMS

_PAIR_A = (0, 0, 0, 1, 1, 2)
_PAIR_B = (1, 2, 3, 2, 3, 3)
_ALPHA = (0.4, 1.0, 0.85, 1.15)


def _route_kernel(ew_ref, pos_ref, coef_ref, meta_ref, mean_ref):
    v = ew_ref[:]
    cols = [v[:, e:e + 1] for e in range(4)]
    sel = []
    ws = []
    for e in range(4):
        rank = jnp.zeros_like(cols[e], dtype=jnp.int32)
        for f in range(4):
            if f == e:
                continue
            if f < e:
                beats = cols[f] >= cols[e]
            else:
                beats = cols[f] > cols[e]
            rank = rank + beats.astype(jnp.int32)
        s = (rank < 2).astype(jnp.float32)
        sel.append(s)
        ws.append(cols[e] * s)
    ssum = ws[0] + ws[1] + ws[2] + ws[3]
    inv = 1.0 / jnp.maximum(ssum, 1e-8)
    w = [wi * inv for wi in ws]
    s_w = w[0] + w[1] + w[2] + w[3]

    coef_ref[:, 0:1] = _ALPHA[0] * w[0]
    coef_ref[:, 1:2] = _ALPHA[1] * w[1]
    coef_ref[:, 2:3] = _ALPHA[2] * w[2]
    coef_ref[:, 3:4] = _ALPHA[3] * w[3]
    coef_ref[:, 4:5] = s_w
    coef_ref[:, 5:6] = 0.15 * w[2]
    coef_ref[:, 6:128] = jnp.zeros((_S, 122), jnp.float32)

    wcat = jnp.concatenate(w, axis=1)
    m = jnp.sum(wcat, axis=0, keepdims=True) / _S
    mrow = jnp.concatenate([m, jnp.zeros((1, 124), jnp.float32)], axis=1)
    mean_ref[:] = jnp.concatenate(
        [mrow, jnp.zeros((7, 128), jnp.float32)], axis=0)

    pairs = [sel[a] * sel[b] for a, b in zip(_PAIR_A, _PAIR_B)]
    ind = jnp.concatenate(pairs + [jnp.zeros((_S, 2), jnp.float32)], axis=1)

    row_i = jax.lax.broadcasted_iota(jnp.int32, (_S, _S), 0)
    col_i = jax.lax.broadcasted_iota(jnp.int32, (_S, _S), 1)
    ltri = (col_i < row_i).astype(jnp.float32).astype(jnp.bfloat16)
    ranks = jnp.dot(ltri, ind.astype(jnp.bfloat16),
                    preferred_element_type=jnp.float32)

    counts = jnp.sum(ind, axis=0, keepdims=True)
    pc = ((counts.astype(jnp.int32) + (_TMS - 1)) // _TMS) * _TMS
    offs = []
    acc = jnp.zeros((1, 1), jnp.int32)
    for p in range(6):
        offs.append(acc)
        acc = acc + pc[:, p:p + 1]
    total = acc
    off_row = jnp.concatenate(
        offs + [jnp.zeros((1, 2), jnp.int32)], axis=1).astype(jnp.float32)

    pos = jnp.sum(ind * (off_row + ranks), axis=1, keepdims=True)
    pos_ref[:] = pos.astype(jnp.int32)

    lane = jax.lax.broadcasted_iota(jnp.int32, (1, 32), 1)
    tp = jnp.zeros((1, 32), jnp.int32)
    for p in range(1, 6):
        tp = tp + (lane * _TMS >= offs[p]).astype(jnp.int32)
    num_tiles = total // _TMS
    meta_ref[:] = jnp.where(lane == 31, num_tiles, tp)


def _gmm_kernel(meta_ref, xc_ref, xm_ref, cs_ref,
                wc_ref, bc_ref, wb_ref, bb_ref,
                wr_ref, br_ref, wd_ref, bd_ref,
                out_ref, wbf_ref, acc_ref):
    n = pl.program_id(0)
    t = pl.program_id(1)
    w_refs = (wc_ref, wb_ref, wr_ref, wd_ref)
    b_refs = (bc_ref, bb_ref, br_ref, bd_ref)

    @pl.when(t == 0)
    def _cast_w():
        for e in range(4):
            wbf_ref[e] = w_refs[e][:].astype(jnp.bfloat16)

    num_tiles = meta_ref[31]

    @pl.when(t < num_tiles)
    def _tile():
        pw = meta_ref[t]
        xc = xc_ref[:]
        xm = xm_ref[:]
        acc_ref[:] = jnp.zeros((_TMS, _TN), jnp.float32)
        for e in range(4):
            in_pairs = [p for p in range(6)
                        if _PAIR_A[p] == e or _PAIR_B[p] == e]
            cond = ((pw == in_pairs[0]) | (pw == in_pairs[1])
                    | (pw == in_pairs[2]))

            @pl.when(cond)
            def _dot(e=e):
                h = jnp.dot(xc, wbf_ref[e, 0:_D, :],
                            preferred_element_type=jnp.float32)
                h = h + jnp.dot(xm, wbf_ref[e, _D:2 * _D, :],
                                preferred_element_type=jnp.float32)
                gate = jax.nn.sigmoid(h + b_refs[e][:])
                acc_ref[:] = acc_ref[:] + cs_ref[:, e:e + 1] * gate

        col = pl.ds(n * _TN, _TN)
        ctx32 = xc_ref[:, col].astype(jnp.float32)
        mem32 = xm_ref[:, col].astype(jnp.float32)
        out_ref[:] = (cs_ref[:, 4:5] * ctx32
                      + (mem32 - ctx32) * acc_ref[:]
                      + cs_ref[:, 5:6] * mem32)


_NW = 32
_BPW = _S // _NW


def _sc_mesh():
    return plsc.VectorSubcoreMesh(
        core_axis_name="c", subcore_axis_name="s", num_cores=2)


def _sc_scatter(ctx_bf, mem_bf, coef, pos):
    dw = _D // 2

    @functools.partial(
        pl.kernel, mesh=_sc_mesh(),
        out_type=[jax.ShapeDtypeStruct((_PAD, dw), jnp.int32),
                  jax.ShapeDtypeStruct((_PAD, dw), jnp.int32),
                  jax.ShapeDtypeStruct((_PAD, 128), jnp.float32)],
        scratch_types=[pltpu.VMEM((_BPW,), jnp.int32),
                       pltpu.VMEM((_BPW, dw), jnp.int32),
                       pltpu.VMEM((_BPW, 128), jnp.float32),
                       pltpu.SemaphoreType.DMA],
    )
    def k(ctx_hbm, mem_hbm, coef_hbm, pos_hbm, xcs_hbm, xms_hbm, cs_hbm,
          idx_v, rows_v, coef_v, sem):
        wid = lax.axis_index("s") * 2 + lax.axis_index("c")
        base = wid * _BPW
        pltpu.sync_copy(pos_hbm.at[pl.ds(base, _BPW)], idx_v)
        pltpu.sync_copy(ctx_hbm.at[pl.ds(base, _BPW)], rows_v)
        pltpu.async_copy(rows_v, xcs_hbm.at[idx_v], sem).wait()
        pltpu.sync_copy(mem_hbm.at[pl.ds(base, _BPW)], rows_v)
        pltpu.async_copy(rows_v, xms_hbm.at[idx_v], sem).wait()
        pltpu.sync_copy(coef_hbm.at[pl.ds(base, _BPW)], coef_v)
        pltpu.async_copy(coef_v, cs_hbm.at[idx_v], sem).wait()

    ctx_i = jax.lax.bitcast_convert_type(
        ctx_bf.reshape(_S, dw, 2), jnp.int32)
    mem_i = jax.lax.bitcast_convert_type(
        mem_bf.reshape(_S, dw, 2), jnp.int32)
    xcs_i, xms_i, cs = k(ctx_i, mem_i, coef, pos)
    xcs = jax.lax.bitcast_convert_type(xcs_i, jnp.bfloat16).reshape(_PAD, _D)
    xms = jax.lax.bitcast_convert_type(xms_i, jnp.bfloat16).reshape(_PAD, _D)
    return xcs, xms, cs


def _sc_gather(out_sorted, pos):
    @functools.partial(
        pl.kernel, mesh=_sc_mesh(),
        out_type=jax.ShapeDtypeStruct((_S, _D), jnp.float32),
        scratch_types=[pltpu.VMEM((_BPW // 2,), jnp.int32),
                       pltpu.VMEM((_BPW // 2, _D), jnp.float32),
                       pltpu.SemaphoreType.DMA],
    )
    def k(outs_hbm, pos_hbm, fused_hbm, idx_v, rows_v, sem):
        wid = lax.axis_index("s") * 2 + lax.axis_index("c")
        base = wid * _BPW
        for c in range(2):
            b2 = base + c * (_BPW // 2)
            pltpu.sync_copy(pos_hbm.at[pl.ds(b2, _BPW // 2)], idx_v)
            pltpu.async_copy(outs_hbm.at[idx_v], rows_v, sem).wait()
            pltpu.sync_copy(rows_v, fused_hbm.at[pl.ds(b2, _BPW // 2)])

    return k(out_sorted, pos)


def _route(ew):
    return pl.pallas_call(
        _route_kernel,
        out_shape=[jax.ShapeDtypeStruct((_S, 1), jnp.int32),
                   jax.ShapeDtypeStruct((_S, 128), jnp.float32),
                   jax.ShapeDtypeStruct((1, 32), jnp.int32),
                   jax.ShapeDtypeStruct((8, 128), jnp.float32)],
    )(ew)


def _gmm(meta, xc_s, xm_s, coef_s, weights, biases):
    n_tiles = _D // _TN
    in_specs = [pl.BlockSpec(memory_space=pltpu.SMEM),
                pl.BlockSpec((_TMS, _D), lambda n, t: (t, 0)),
                pl.BlockSpec((_TMS, _D), lambda n, t: (t, 0)),
                pl.BlockSpec((_TMS, 128), lambda n, t: (t, 0))]
    operands = [meta, xc_s, xm_s, coef_s]
    for W, b in zip(weights, biases):
        in_specs += [pl.BlockSpec((2 * _D, _TN), lambda n, t: (0, n)),
                     pl.BlockSpec((1, _TN), lambda n, t: (0, n))]
        operands += [W, b]
    return pl.pallas_call(
        _gmm_kernel,
        grid=(n_tiles, _TMAX),
        in_specs=in_specs,
        out_specs=pl.BlockSpec((_TMS, _TN), lambda n, t: (t, n)),
        out_shape=jax.ShapeDtypeStruct((_PAD, _D), jnp.float32),
        scratch_shapes=[pltpu.VMEM((4, 2 * _D, _TN), jnp.bfloat16),
                        pltpu.VMEM((_TMS, _TN), jnp.float32)],
        compiler_params=pltpu.CompilerParams(
            dimension_semantics=("arbitrary", "arbitrary")),
    )(*operands)


@jax.jit
def kernel(context_state, memory_state, expert_weights,
           W_conservative, b_conservative, W_base, b_base,
           W_bridge, b_bridge, W_memory_dominant, b_memory_dominant):
    B, S, d = context_state.shape
    ctx = context_state.reshape(S, d).astype(jnp.bfloat16)
    mem = memory_state.reshape(S, d).astype(jnp.bfloat16)
    ew = expert_weights.reshape(S, 4)
    biases = [b.reshape(1, d) for b in (b_conservative, b_base, b_bridge,
                                        b_memory_dominant)]
    weights = [W_conservative, W_base, W_bridge, W_memory_dominant]

    pos, coef, meta, mean_pad = _route(ew)
    p = pos.reshape(_S)

    xc_s, xm_s, coef_s = _sc_scatter(ctx, mem, coef, p)
    out_sorted = _gmm(meta.reshape(32), xc_s, xm_s, coef_s, weights, biases)
    fused = _sc_gather(out_sorted, p).reshape(B, S, d)
    mean_weights = mean_pad[0, 0:4]
    return fused, mean_weights


# speedup vs baseline: 1.5967x; 1.5967x over previous
import jax
import jax.numpy as jnp
from jax.experimental import pallas as pl
from jax.experimental.pallas import tpu as pltpu

_S = 2048
_D = 2048
_TN = 256
_TMS = 128
_TMAX = 21
_PAD = _TMAX * _TMS

_PAIR_A = (0, 0, 0, 1, 1, 2)
_PAIR_B = (1, 2, 3, 2, 3, 3)
_ALPHA = (0.4, 1.0, 0.85, 1.15)


def _route_kernel(ew_ref, pos_ref, tok_ref, coef_ref, meta_ref, mean_ref):
    v = ew_ref[:]
    cols = [v[:, e:e + 1] for e in range(4)]
    sel = []
    ws = []
    for e in range(4):
        rank = jnp.zeros_like(cols[e], dtype=jnp.int32)
        for f in range(4):
            if f == e:
                continue
            if f < e:
                beats = cols[f] >= cols[e]
            else:
                beats = cols[f] > cols[e]
            rank = rank + beats.astype(jnp.int32)
        s = (rank < 2).astype(jnp.float32)
        sel.append(s)
        ws.append(cols[e] * s)
    ssum = ws[0] + ws[1] + ws[2] + ws[3]
    inv = 1.0 / jnp.maximum(ssum, 1e-8)
    w = [wi * inv for wi in ws]
    s_w = w[0] + w[1] + w[2] + w[3]

    coef_ref[:, 0:1] = _ALPHA[0] * w[0]
    coef_ref[:, 1:2] = _ALPHA[1] * w[1]
    coef_ref[:, 2:3] = _ALPHA[2] * w[2]
    coef_ref[:, 3:4] = _ALPHA[3] * w[3]
    coef_ref[:, 4:5] = s_w
    coef_ref[:, 5:6] = 0.15 * w[2]
    coef_ref[:, 6:8] = jnp.zeros((_S, 2), jnp.float32)

    wcat = jnp.concatenate(w, axis=1)
    m = jnp.sum(wcat, axis=0, keepdims=True) / _S
    mrow = jnp.concatenate([m, jnp.zeros((1, 124), jnp.float32)], axis=1)
    mean_ref[:] = jnp.concatenate(
        [mrow, jnp.zeros((7, 128), jnp.float32)], axis=0)

    pairs = [sel[a] * sel[b] for a, b in zip(_PAIR_A, _PAIR_B)]
    ind = jnp.concatenate(pairs + [jnp.zeros((_S, 2), jnp.float32)], axis=1)

    row_i = jax.lax.broadcasted_iota(jnp.int32, (_S, _S), 0)
    col_i = jax.lax.broadcasted_iota(jnp.int32, (_S, _S), 1)
    ltri = (col_i < row_i).astype(jnp.float32).astype(jnp.bfloat16)
    ranks = jnp.dot(ltri, ind.astype(jnp.bfloat16),
                    preferred_element_type=jnp.float32)

    counts = jnp.sum(ind, axis=0, keepdims=True)
    pc = ((counts.astype(jnp.int32) + (_TMS - 1)) // _TMS) * _TMS
    offs = []
    acc = jnp.zeros((1, 1), jnp.int32)
    for p in range(6):
        offs.append(acc)
        acc = acc + pc[:, p:p + 1]
    total = acc
    off_row = jnp.concatenate(
        offs + [jnp.zeros((1, 2), jnp.int32)], axis=1).astype(jnp.float32)

    pos = jnp.sum(ind * (off_row + ranks), axis=1, keepdims=True)
    pos_i = pos.astype(jnp.int32)
    pos_ref[:] = pos_i

    col_s = jax.lax.broadcasted_iota(jnp.int32, (_S, _PAD), 1)
    perm_t = (col_s == pos_i).astype(jnp.float32).astype(jnp.bfloat16)
    tid = jax.lax.broadcasted_iota(jnp.int32, (_S, 8), 0)
    lane8 = jax.lax.broadcasted_iota(jnp.int32, (_S, 8), 1)
    hi = (tid // 256).astype(jnp.float32)
    lo = (tid % 256).astype(jnp.float32)
    hl = jnp.where(lane8 == 0, hi, jnp.where(lane8 == 1, lo, 0.0))
    tok2 = jax.lax.dot_general(
        perm_t, hl.astype(jnp.bfloat16),
        dimension_numbers=(((0,), (0,)), ((), ())),
        preferred_element_type=jnp.float32)
    tok_ref[:] = (tok2[:, 0:1] * 256.0 + tok2[:, 1:2]).astype(jnp.int32)

    lane = jax.lax.broadcasted_iota(jnp.int32, (1, 32), 1)
    tp = jnp.zeros((1, 32), jnp.int32)
    for p in range(1, 6):
        tp = tp + (lane * _TMS >= offs[p]).astype(jnp.int32)
    num_tiles = total // _TMS
    meta_ref[:] = jnp.where(lane == 31, num_tiles, tp)


def _gmm_kernel(meta_ref, xs_ref, cs_ref,
                wc_ref, bc_ref, wb_ref, bb_ref,
                wr_ref, br_ref, wd_ref, bd_ref,
                out_ref, wbf_ref, acc_ref):
    n = pl.program_id(0)
    t = pl.program_id(1)
    w_refs = (wc_ref, wb_ref, wr_ref, wd_ref)
    b_refs = (bc_ref, bb_ref, br_ref, bd_ref)

    @pl.when(t == 0)
    def _cast_w():
        for e in range(4):
            wbf_ref[e] = w_refs[e][:].astype(jnp.bfloat16)

    num_tiles = meta_ref[31]

    @pl.when(t < num_tiles)
    def _tile():
        pw = meta_ref[t]
        xc = xs_ref[:, 0:_D]
        xm = xs_ref[:, _D:2 * _D]
        acc_ref[:] = jnp.zeros((_TMS, _TN), jnp.float32)
        for e in range(4):
            in_pairs = [p for p in range(6)
                        if _PAIR_A[p] == e or _PAIR_B[p] == e]
            cond = ((pw == in_pairs[0]) | (pw == in_pairs[1])
                    | (pw == in_pairs[2]))

            @pl.when(cond)
            def _dot(e=e):
                h = jnp.dot(xc, wbf_ref[e, 0:_D, :],
                            preferred_element_type=jnp.float32)
                h = h + jnp.dot(xm, wbf_ref[e, _D:2 * _D, :],
                                preferred_element_type=jnp.float32)
                gate = jax.nn.sigmoid(h + b_refs[e][:])
                acc_ref[:] = acc_ref[:] + cs_ref[:, e:e + 1] * gate

        col = pl.ds(n * _TN, _TN)
        ctx32 = xs_ref[:, col].astype(jnp.float32)
        mcol = pl.ds(_D + n * _TN, _TN)
        mem32 = xs_ref[:, mcol].astype(jnp.float32)
        out_ref[:] = (cs_ref[:, 4:5] * ctx32
                      + (mem32 - ctx32) * acc_ref[:]
                      + cs_ref[:, 5:6] * mem32)


def _route(ew):
    return pl.pallas_call(
        _route_kernel,
        out_shape=[jax.ShapeDtypeStruct((_S, 1), jnp.int32),
                   jax.ShapeDtypeStruct((_PAD, 1), jnp.int32),
                   jax.ShapeDtypeStruct((_S, 8), jnp.float32),
                   jax.ShapeDtypeStruct((1, 32), jnp.int32),
                   jax.ShapeDtypeStruct((8, 128), jnp.float32)],
    )(ew)


def _gmm(meta, xs_s, coef_s, weights, biases):
    n_tiles = _D // _TN
    in_specs = [pl.BlockSpec(memory_space=pltpu.SMEM),
                pl.BlockSpec((_TMS, 2 * _D), lambda n, t: (t, 0)),
                pl.BlockSpec((_TMS, 8), lambda n, t: (t, 0))]
    operands = [meta, xs_s, coef_s]
    for W, b in zip(weights, biases):
        in_specs += [pl.BlockSpec((2 * _D, _TN), lambda n, t: (0, n)),
                     pl.BlockSpec((1, _TN), lambda n, t: (0, n))]
        operands += [W, b]
    return pl.pallas_call(
        _gmm_kernel,
        grid=(n_tiles, _TMAX),
        in_specs=in_specs,
        out_specs=pl.BlockSpec((_TMS, _TN), lambda n, t: (t, n)),
        out_shape=jax.ShapeDtypeStruct((_PAD, _D), jnp.float32),
        scratch_shapes=[pltpu.VMEM((4, 2 * _D, _TN), jnp.bfloat16),
                        pltpu.VMEM((_TMS, _TN), jnp.float32)],
        compiler_params=pltpu.CompilerParams(
            dimension_semantics=("arbitrary", "arbitrary")),
    )(*operands)


@jax.jit
def kernel(context_state, memory_state, expert_weights,
           W_conservative, b_conservative, W_base, b_base,
           W_bridge, b_bridge, W_memory_dominant, b_memory_dominant):
    B, S, d = context_state.shape
    x_bf = jnp.concatenate(
        [context_state.reshape(S, d), memory_state.reshape(S, d)],
        axis=1).astype(jnp.bfloat16)
    ew = expert_weights.reshape(S, 4)
    biases = [b.reshape(1, d) for b in (b_conservative, b_base, b_bridge,
                                        b_memory_dominant)]
    weights = [W_conservative, W_base, W_bridge, W_memory_dominant]

    pos, tok, coef, meta, mean_pad = _route(ew)
    p = pos.reshape(_S)
    t = tok.reshape(_PAD)

    xs_s = jnp.take(x_bf, t, axis=0)
    coef_s = jnp.take(coef, t, axis=0)

    out_sorted = _gmm(meta.reshape(32), xs_s, coef_s, weights, biases)

    fused = jnp.take(out_sorted, p, axis=0).reshape(B, S, d)
    mean_weights = mean_pad[0, 0:4]
    return fused, mean_weights


# speedup vs baseline: 2.8261x; 1.7700x over previous
import jax
import jax.numpy as jnp
from jax.experimental import pallas as pl
from jax.experimental.pallas import tpu as pltpu

_S = 2048
_D = 2048
_TN = 256
_TM = 512


def _fused_kernel(ctx_ref, mem_ref, ew_ref,
                  wc_ref, bc_ref, wb_ref, bb_ref,
                  wr_ref, br_ref, wd_ref, bd_ref,
                  out_ref, mean_ref, coef_ref):
    n = pl.program_id(0)

    @pl.when(n == 0)
    def _routing():
        v = ew_ref[:]
        cols = [v[:, e:e + 1] for e in range(4)]
        ws = []
        for e in range(4):
            rank = jnp.zeros_like(cols[e], dtype=jnp.int32)
            for f in range(4):
                if f == e:
                    continue
                if f < e:
                    beats = cols[f] >= cols[e]
                else:
                    beats = cols[f] > cols[e]
                rank = rank + beats.astype(jnp.int32)
            sel = (rank < 2).astype(jnp.float32)
            ws.append(cols[e] * sel)
        s = ws[0] + ws[1] + ws[2] + ws[3]
        inv = 1.0 / jnp.maximum(s, 1e-8)
        w = [wi * inv for wi in ws]
        alphas = (0.4, 1.0, 0.85, 1.15)
        s_w = w[0] + w[1] + w[2] + w[3]
        coef_ref[:, 0:1] = alphas[0] * w[0]
        coef_ref[:, 1:2] = alphas[1] * w[1]
        coef_ref[:, 2:3] = alphas[2] * w[2]
        coef_ref[:, 3:4] = alphas[3] * w[3]
        coef_ref[:, 4:5] = s_w
        coef_ref[:, 5:6] = 0.15 * w[2]
        coef_ref[:, 6:8] = jnp.zeros((_S, 2), jnp.float32)
        wcat = jnp.concatenate(w, axis=1)
        m = jnp.sum(wcat, axis=0, keepdims=True) / _S
        mrow = jnp.concatenate([m, jnp.zeros((1, 124), jnp.float32)], axis=1)
        mean_ref[:] = jnp.concatenate(
            [mrow, jnp.zeros((7, 128), jnp.float32)], axis=0)

    w_refs = (wc_ref, wb_ref, wr_ref, wd_ref)
    b_refs = (bc_ref, bb_ref, br_ref, bd_ref)
    wtop = [w_refs[e][0:_D, :] for e in range(4)]
    wbot = [w_refs[e][_D:2 * _D, :] for e in range(4)]

    col = pl.ds(n * _TN, _TN)
    nchunk = _S // _TM
    for i in range(nchunk):
        rows = pl.ds(i * _TM, _TM)
        xc = ctx_ref[rows, :]
        xm = mem_ref[rows, :]
        acc = jnp.zeros((_TM, _TN), jnp.float32)
        for e in range(4):
            h = jnp.dot(xc, wtop[e], preferred_element_type=jnp.float32)
            h = h + jnp.dot(xm, wbot[e], preferred_element_type=jnp.float32)
            gate = jax.nn.sigmoid(h + b_refs[e][:])
            acc = acc + coef_ref[rows, e:e + 1] * gate
        ctx32 = ctx_ref[rows, col].astype(jnp.float32)
        mem32 = mem_ref[rows, col].astype(jnp.float32)
        out_ref[rows, :] = (coef_ref[rows, 4:5] * ctx32
                            + (mem32 - ctx32) * acc
                            + coef_ref[rows, 5:6] * mem32)


@jax.jit
def kernel(context_state, memory_state, expert_weights,
           W_conservative, b_conservative, W_base, b_base,
           W_bridge, b_bridge, W_memory_dominant, b_memory_dominant):
    B, S, d = context_state.shape
    ctx = context_state.reshape(S, d).astype(jnp.bfloat16)
    mem = memory_state.reshape(S, d).astype(jnp.bfloat16)
    ew = expert_weights.reshape(S, 4)
    biases = [b.reshape(1, d) for b in (b_conservative, b_base, b_bridge,
                                        b_memory_dominant)]
    weights = [W.astype(jnp.bfloat16) for W in
               (W_conservative, W_base, W_bridge, W_memory_dominant)]

    n_tiles = d // _TN
    full = lambda n: (0, 0)
    wspec = pl.BlockSpec((2 * d, _TN), lambda n: (0, n))
    bspec = pl.BlockSpec((1, _TN), lambda n: (0, n))

    in_specs = [pl.BlockSpec((S, d), full),
                pl.BlockSpec((S, d), full),
                pl.BlockSpec((S, 4), full)]
    operands = [ctx, mem, ew]
    for W, b in zip(weights, biases):
        in_specs += [wspec, bspec]
        operands += [W, b]

    out, mean_pad = pl.pallas_call(
        _fused_kernel,
        grid=(n_tiles,),
        in_specs=in_specs,
        out_specs=[pl.BlockSpec((S, _TN), lambda n: (0, n)),
                   pl.BlockSpec((8, 128), full)],
        out_shape=[jax.ShapeDtypeStruct((S, d), jnp.float32),
                   jax.ShapeDtypeStruct((8, 128), jnp.float32)],
        scratch_shapes=[pltpu.VMEM((S, 8), jnp.float32)],
        compiler_params=pltpu.CompilerParams(
            dimension_semantics=("arbitrary",)),
    )(*operands)

    fused = out.reshape(B, S, d)
    mean_weights = mean_pad[0, 0:4]
    return fused, mean_weights


# speedup vs baseline: 3.6060x; 1.2759x over previous
import jax
import jax.numpy as jnp
from jax.experimental import pallas as pl
from jax.experimental.pallas import tpu as pltpu

_S = 2048
_D = 2048
_TN = 256
_TM = 512


def _fused_kernel(ctx_ref, mem_ref, ew_ref,
                  wc_ref, bc_ref, wb_ref, bb_ref,
                  wr_ref, br_ref, wd_ref, bd_ref,
                  out_ref, mean_ref, coef_ref, acc_ref):
    n = pl.program_id(0)
    k2 = pl.program_id(1)

    @pl.when((n == 0) & (k2 == 0))
    def _routing():
        v = ew_ref[:]
        cols = [v[:, e:e + 1] for e in range(4)]
        ws = []
        for e in range(4):
            rank = jnp.zeros_like(cols[e], dtype=jnp.int32)
            for f in range(4):
                if f == e:
                    continue
                if f < e:
                    beats = cols[f] >= cols[e]
                else:
                    beats = cols[f] > cols[e]
                rank = rank + beats.astype(jnp.int32)
            sel = (rank < 2).astype(jnp.float32)
            ws.append(cols[e] * sel)
        s = ws[0] + ws[1] + ws[2] + ws[3]
        inv = 1.0 / jnp.maximum(s, 1e-8)
        w = [wi * inv for wi in ws]
        alphas = (0.4, 1.0, 0.85, 1.15)
        s_w = w[0] + w[1] + w[2] + w[3]
        coef_ref[:, 0:1] = alphas[0] * w[0]
        coef_ref[:, 1:2] = alphas[1] * w[1]
        coef_ref[:, 2:3] = alphas[2] * w[2]
        coef_ref[:, 3:4] = alphas[3] * w[3]
        coef_ref[:, 4:5] = s_w
        coef_ref[:, 5:6] = 0.15 * w[2]
        coef_ref[:, 6:8] = jnp.zeros((_S, 2), jnp.float32)
        wcat = jnp.concatenate(w, axis=1)
        m = jnp.sum(wcat, axis=0, keepdims=True) / _S
        mrow = jnp.concatenate([m, jnp.zeros((1, 124), jnp.float32)], axis=1)
        mean_ref[:] = jnp.concatenate(
            [mrow, jnp.zeros((7, 128), jnp.float32)], axis=0)

    w_refs = (wc_ref, wb_ref, wr_ref, wd_ref)
    b_refs = (bc_ref, bb_ref, br_ref, bd_ref)
    wblk = [w_refs[e][:].astype(jnp.bfloat16) for e in range(4)]

    col = pl.ds(n * _TN, _TN)
    nchunk = _S // _TM

    @pl.when(k2 == 0)
    def _k_ctx():
        for i in range(nchunk):
            rows = pl.ds(i * _TM, _TM)
            xb = ctx_ref[rows, :]
            for e in range(4):
                acc_ref[pl.ds(i * _TM + e * _S, _TM), :] = jnp.dot(
                    xb, wblk[e],
                    preferred_element_type=jnp.float32).astype(jnp.bfloat16)

    @pl.when(k2 == 1)
    def _k_mem_epilogue():
        for i in range(nchunk):
            rows = pl.ds(i * _TM, _TM)
            xb = mem_ref[rows, :]
            acc = jnp.zeros((_TM, _TN), jnp.float32)
            for e in range(4):
                h = acc_ref[pl.ds(i * _TM + e * _S, _TM), :].astype(
                    jnp.float32)
                h = h + jnp.dot(xb, wblk[e],
                                preferred_element_type=jnp.float32)
                gate = jax.nn.sigmoid(h + b_refs[e][:])
                acc = acc + coef_ref[rows, e:e + 1] * gate
            ctx32 = ctx_ref[rows, col].astype(jnp.float32)
            mem32 = mem_ref[rows, col].astype(jnp.float32)
            out_ref[rows, :] = (coef_ref[rows, 4:5] * ctx32
                                + (mem32 - ctx32) * acc
                                + coef_ref[rows, 5:6] * mem32)


@jax.jit
def kernel(context_state, memory_state, expert_weights,
           W_conservative, b_conservative, W_base, b_base,
           W_bridge, b_bridge, W_memory_dominant, b_memory_dominant):
    B, S, d = context_state.shape
    ctx = context_state.reshape(S, d).astype(jnp.bfloat16)
    mem = memory_state.reshape(S, d).astype(jnp.bfloat16)
    ew = expert_weights.reshape(S, 4)
    biases = [b.reshape(1, d) for b in (b_conservative, b_base, b_bridge,
                                        b_memory_dominant)]
    weights = [W_conservative, W_base, W_bridge, W_memory_dominant]

    n_tiles = d // _TN
    full = lambda n, k: (0, 0)
    wspec = pl.BlockSpec((d, _TN), lambda n, k: (k, n))
    bspec = pl.BlockSpec((1, _TN), lambda n, k: (0, n))

    in_specs = [pl.BlockSpec((S, d), full),
                pl.BlockSpec((S, d), full),
                pl.BlockSpec((S, 4), full)]
    operands = [ctx, mem, ew]
    for W, b in zip(weights, biases):
        in_specs += [wspec, bspec]
        operands += [W, b]

    out, mean_pad = pl.pallas_call(
        _fused_kernel,
        grid=(n_tiles, 2),
        in_specs=in_specs,
        out_specs=[pl.BlockSpec((S, _TN), lambda n, k: (0, n)),
                   pl.BlockSpec((8, 128), full)],
        out_shape=[jax.ShapeDtypeStruct((S, d), jnp.float32),
                   jax.ShapeDtypeStruct((8, 128), jnp.float32)],
        scratch_shapes=[pltpu.VMEM((S, 8), jnp.float32),
                        pltpu.VMEM((4 * _S, _TN), jnp.bfloat16)],
        compiler_params=pltpu.CompilerParams(
            dimension_semantics=("arbitrary", "arbitrary")),
    )(*operands)

    fused = out.reshape(B, S, d)
    mean_weights = mean_pad[0, 0:4]
    return fused, mean_weights
